# baseline (speedup 1.0000x reference)
import functools

import jax
import jax.numpy as jnp
from jax import lax
from jax.experimental import pallas as pl
from jax.experimental.pallas import tpu as pltpu
from jax.experimental.pallas import tpu_sc as plsc

B = 1024
T = 128
OBS = 256
A = 128

_BB = 64
_NW = 32
_ROWS = B // _NW
_MCH = 1024

_vector_mesh = plsc.VectorSubcoreMesh(core_axis_name="c", subcore_axis_name="s")


def _obs_kernel(step_ref, obs_in, obs_new, obs_out):
    s = step_ref[0]
    t3 = jax.lax.broadcasted_iota(jnp.int32, (1, T, 1), 1)
    obs_out[...] = jnp.where(t3 == s, obs_new[...][:, None, :], obs_in[...])


def _fill_idx(idx_vmem, base, stride, s_vec):
    iot = lax.iota(jnp.int32, 16)
    idx_vmem[pl.ds(0, 16)] = (base + iot) * stride + s_vec
    idx_vmem[pl.ds(16, 16)] = (base + 16 + iot) * stride + s_vec


def _sc_store_body(mask_in, act_in, rew_in, lp_in, val_in,
                   am_new, a_new, r_new, l_new, v_new, step_in,
                   mask_out, act_out, rew_out, lp_out, val_out,
                   step_v, idx_t, idx_v, am_v, upd_v, upd_i,
                   act_st, f32_st, sem):
    wid = lax.axis_index("s") * 2 + lax.axis_index("c")
    base = wid * _ROWS
    pltpu.async_copy(step_in, step_v, sem).wait()
    s_vec = step_v[...]
    rows = pl.ds(base, _ROWS)
    flat_t = pl.ds(base * T, _ROWS * T)
    flat_v = pl.ds(base * (T + 1), _ROWS * (T + 1))
    mrows = pl.ds(base * T, _ROWS * T)

    pltpu.async_copy(mask_in.at[mrows], mask_out.at[mrows], sem).wait()
    pltpu.async_copy(act_in.at[flat_t], act_st, sem).wait()
    pltpu.async_copy(act_st, act_out.at[flat_t], sem).wait()
    pltpu.async_copy(rew_in.at[flat_t], f32_st.at[pl.ds(0, _ROWS * T)], sem).wait()
    pltpu.async_copy(f32_st.at[pl.ds(0, _ROWS * T)], rew_out.at[flat_t], sem).wait()
    pltpu.async_copy(lp_in.at[flat_t], f32_st.at[pl.ds(0, _ROWS * T)], sem).wait()
    pltpu.async_copy(f32_st.at[pl.ds(0, _ROWS * T)], lp_out.at[flat_t], sem).wait()
    pltpu.async_copy(val_in.at[flat_v], f32_st, sem).wait()
    pltpu.async_copy(f32_st, val_out.at[flat_v], sem).wait()

    _fill_idx(idx_t, base, T, s_vec)
    _fill_idx(idx_v, base, T + 1, s_vec)

    pltpu.async_copy(am_new.at[rows], am_v, sem).wait()
    pltpu.async_copy(am_v, mask_out.at[idx_t], sem).wait()
    pltpu.async_copy(a_new.at[rows], upd_i, sem).wait()
    pltpu.async_copy(upd_i, act_out.at[idx_t], sem).wait()
    pltpu.async_copy(r_new.at[rows], upd_v, sem).wait()
    pltpu.async_copy(upd_v, rew_out.at[idx_t], sem).wait()
    pltpu.async_copy(l_new.at[rows], upd_v, sem).wait()
    pltpu.async_copy(upd_v, lp_out.at[idx_t], sem).wait()
    pltpu.async_copy(v_new.at[rows], upd_v, sem).wait()
    pltpu.async_copy(upd_v, val_out.at[idx_v], sem).wait()


_sc_store = functools.partial(
    pl.kernel,
    out_type=(
        jax.ShapeDtypeStruct((B * T, A), jnp.bool_),
        jax.ShapeDtypeStruct((B * T,), jnp.int32),
        jax.ShapeDtypeStruct((B * T,), jnp.float32),
        jax.ShapeDtypeStruct((B * T,), jnp.float32),
        jax.ShapeDtypeStruct((B * (T + 1),), jnp.float32),
    ),
    mesh=_vector_mesh,
    scratch_types=[
        pltpu.VMEM((16,), jnp.int32),
        pltpu.VMEM((_ROWS,), jnp.int32),
        pltpu.VMEM((_ROWS,), jnp.int32),
        pltpu.VMEM((_ROWS, A), jnp.bool_),
        pltpu.VMEM((_ROWS,), jnp.float32),
        pltpu.VMEM((_ROWS,), jnp.int32),
        pltpu.VMEM((_ROWS * T,), jnp.int32),
        pltpu.VMEM((_ROWS * (T + 1),), jnp.float32),
        pltpu.SemaphoreType.DMA,
    ],
)(_sc_store_body)


def kernel(state_obs, state_action_mask, state_actions, state_rewards,
           state_log_prob, state_values, state_advantages, state_targets,
           step, obs, action_mask, action, reward, log_prob, value):
    step_arr = jnp.asarray(step, jnp.int32).reshape((1,))

    new_obs = pl.pallas_call(
        _obs_kernel,
        grid=(B // _BB,),
        in_specs=[
            pl.BlockSpec(memory_space=pltpu.SMEM),
            pl.BlockSpec((_BB, T, OBS), lambda i: (i, 0, 0)),
            pl.BlockSpec((_BB, OBS), lambda i: (i, 0)),
        ],
        out_specs=pl.BlockSpec((_BB, T, OBS), lambda i: (i, 0, 0)),
        out_shape=jax.ShapeDtypeStruct((B, T, OBS), jnp.float32),
    )(step_arr, state_obs, obs)

    new_mask, new_act, new_rew, new_lp, new_val = _sc_store(
        state_action_mask.reshape(B * T, A),
        state_actions.reshape(B * T),
        state_rewards.reshape(B * T),
        state_log_prob.reshape(B * T),
        state_values.reshape(B * (T + 1)),
        action_mask, action, reward, log_prob, value,
        jnp.full((16,), jnp.asarray(step, jnp.int32), jnp.int32))

    return (new_obs,
            new_mask.reshape(B, T, A),
            new_act.reshape(B, T),
            new_rew.reshape(B, T),
            new_lp.reshape(B, T),
            new_val.reshape(B, T + 1),
            state_advantages, state_targets)


# speedup vs baseline: 10.2842x; 10.2842x over previous
import functools

import jax
import jax.numpy as jnp
from jax import lax
from jax.experimental import pallas as pl
from jax.experimental.pallas import tpu as pltpu
from jax.experimental.pallas import tpu_sc as plsc

B = 1024
T = 128
OBS = 256
A = 128

_BB = 64
_NW = 32
_ROWS = B // _NW

_vector_mesh = plsc.VectorSubcoreMesh(core_axis_name="c", subcore_axis_name="s")


def _tc_kernel(step_ref, obs_in, mask_in, obs_new, mask_new,
               obs_out, mask_out):
    s = step_ref[0]
    t3 = jax.lax.broadcasted_iota(jnp.int32, (1, T, 1), 1)
    hit3 = t3 == s
    obs_out[...] = jnp.where(hit3, obs_new[...][:, None, :], obs_in[...])
    m_in = mask_in[...].astype(jnp.int8)
    m_new = mask_new[...].astype(jnp.int8)[:, None, :]
    mask_out[...] = jnp.where(hit3, m_new, m_in) != 0


def _fill_idx(idx_vmem, base, stride, s_vec):
    iot = lax.iota(jnp.int32, 16)
    idx_vmem[pl.ds(0, 16)] = (base + iot) * stride + s_vec
    idx_vmem[pl.ds(16, 16)] = (base + 16 + iot) * stride + s_vec


def _sc_store_body(act_in, rew_in, lp_in, val_in,
                   a_new, r_new, l_new, v_new, step_in,
                   act_out, rew_out, lp_out, val_out,
                   step_v, idx_t, idx_v, upd_v, upd_i,
                   act_st, f32_st, sem):
    wid = lax.axis_index("s") * 2 + lax.axis_index("c")
    base = wid * _ROWS
    pltpu.async_copy(step_in, step_v, sem).wait()
    s_vec = step_v[...]
    rows = pl.ds(base, _ROWS)
    flat_t = pl.ds(base * T, _ROWS * T)
    flat_v = pl.ds(base * (T + 1), _ROWS * (T + 1))

    pltpu.async_copy(act_in.at[flat_t], act_st, sem).wait()
    pltpu.async_copy(act_st, act_out.at[flat_t], sem).wait()
    pltpu.async_copy(rew_in.at[flat_t], f32_st.at[pl.ds(0, _ROWS * T)], sem).wait()
    pltpu.async_copy(f32_st.at[pl.ds(0, _ROWS * T)], rew_out.at[flat_t], sem).wait()
    pltpu.async_copy(lp_in.at[flat_t], f32_st.at[pl.ds(0, _ROWS * T)], sem).wait()
    pltpu.async_copy(f32_st.at[pl.ds(0, _ROWS * T)], lp_out.at[flat_t], sem).wait()
    pltpu.async_copy(val_in.at[flat_v], f32_st, sem).wait()
    pltpu.async_copy(f32_st, val_out.at[flat_v], sem).wait()

    _fill_idx(idx_t, base, T, s_vec)
    _fill_idx(idx_v, base, T + 1, s_vec)

    pltpu.async_copy(a_new.at[rows], upd_i, sem).wait()
    pltpu.async_copy(upd_i, act_out.at[idx_t], sem).wait()
    pltpu.async_copy(r_new.at[rows], upd_v, sem).wait()
    pltpu.async_copy(upd_v, rew_out.at[idx_t], sem).wait()
    pltpu.async_copy(l_new.at[rows], upd_v, sem).wait()
    pltpu.async_copy(upd_v, lp_out.at[idx_t], sem).wait()
    pltpu.async_copy(v_new.at[rows], upd_v, sem).wait()
    pltpu.async_copy(upd_v, val_out.at[idx_v], sem).wait()


_sc_store = functools.partial(
    pl.kernel,
    out_type=(
        jax.ShapeDtypeStruct((B * T,), jnp.int32),
        jax.ShapeDtypeStruct((B * T,), jnp.float32),
        jax.ShapeDtypeStruct((B * T,), jnp.float32),
        jax.ShapeDtypeStruct((B * (T + 1),), jnp.float32),
    ),
    mesh=_vector_mesh,
    scratch_types=[
        pltpu.VMEM((16,), jnp.int32),
        pltpu.VMEM((_ROWS,), jnp.int32),
        pltpu.VMEM((_ROWS,), jnp.int32),
        pltpu.VMEM((_ROWS,), jnp.float32),
        pltpu.VMEM((_ROWS,), jnp.int32),
        pltpu.VMEM((_ROWS * T,), jnp.int32),
        pltpu.VMEM((_ROWS * (T + 1),), jnp.float32),
        pltpu.SemaphoreType.DMA,
    ],
)(_sc_store_body)


def kernel(state_obs, state_action_mask, state_actions, state_rewards,
           state_log_prob, state_values, state_advantages, state_targets,
           step, obs, action_mask, action, reward, log_prob, value):
    step_arr = jnp.asarray(step, jnp.int32).reshape((1,))

    new_obs, new_mask = pl.pallas_call(
        _tc_kernel,
        grid=(B // _BB,),
        in_specs=[
            pl.BlockSpec(memory_space=pltpu.SMEM),
            pl.BlockSpec((_BB, T, OBS), lambda i: (i, 0, 0)),
            pl.BlockSpec((_BB, T, A), lambda i: (i, 0, 0)),
            pl.BlockSpec((_BB, OBS), lambda i: (i, 0)),
            pl.BlockSpec((_BB, A), lambda i: (i, 0)),
        ],
        out_specs=[
            pl.BlockSpec((_BB, T, OBS), lambda i: (i, 0, 0)),
            pl.BlockSpec((_BB, T, A), lambda i: (i, 0, 0)),
        ],
        out_shape=(
            jax.ShapeDtypeStruct((B, T, OBS), jnp.float32),
            jax.ShapeDtypeStruct((B, T, A), jnp.bool_),
        ),
    )(step_arr, state_obs, state_action_mask, obs, action_mask)

    new_act, new_rew, new_lp, new_val = _sc_store(
        state_actions.reshape(B * T),
        state_rewards.reshape(B * T),
        state_log_prob.reshape(B * T),
        state_values.reshape(B * (T + 1)),
        action, reward, log_prob, value,
        jnp.full((16,), jnp.asarray(step, jnp.int32), jnp.int32))

    return (new_obs, new_mask,
            new_act.reshape(B, T),
            new_rew.reshape(B, T),
            new_lp.reshape(B, T),
            new_val.reshape(B, T + 1),
            state_advantages, state_targets)


# speedup vs baseline: 10.3212x; 1.0036x over previous
import functools

import jax
import jax.numpy as jnp
from jax import lax
from jax.experimental import pallas as pl
from jax.experimental.pallas import tpu as pltpu
from jax.experimental.pallas import tpu_sc as plsc

B = 1024
T = 128
OBS = 256
A = 128

_BB = 64
_NW = 32
_ROWS = B // _NW

_vector_mesh = plsc.VectorSubcoreMesh(core_axis_name="c", subcore_axis_name="s")


def _tc_kernel(step_ref, obs_in, mask_in, obs_new, mask_new,
               obs_out, mask_out):
    s = step_ref[0]
    t3 = jax.lax.broadcasted_iota(jnp.int32, (1, T, 1), 1)
    hit3 = t3 == s
    obs_out[...] = jnp.where(hit3, obs_new[...][:, None, :], obs_in[...])
    m_in = mask_in[...].astype(jnp.int8)
    m_new = mask_new[...].astype(jnp.int8)[:, None, :]
    mask_out[...] = jnp.where(hit3, m_new, m_in) != 0


def _fill_idx(idx_vmem, base, stride, s_vec):
    iot = lax.iota(jnp.int32, 16)
    idx_vmem[pl.ds(0, 16)] = (base + iot) * stride + s_vec
    idx_vmem[pl.ds(16, 16)] = (base + 16 + iot) * stride + s_vec


def _sc_store_body(act_in, rew_in, lp_in, val_in,
                   a_new, r_new, l_new, v_new, step_in,
                   act_out, rew_out, lp_out, val_out,
                   step_v, idx_t, idx_v, upd_v, upd_i,
                   act_st, f32_st, sem):
    wid = lax.axis_index("s") * 2 + lax.axis_index("c")
    base = wid * _ROWS
    pltpu.async_copy(step_in, step_v, sem).wait()
    s_vec = step_v[...]
    rows = pl.ds(base, _ROWS)
    flat_t = pl.ds(base * T, _ROWS * T)
    flat_v = pl.ds(base * (T + 1), _ROWS * (T + 1))

    pltpu.async_copy(act_in.at[flat_t], act_st, sem).wait()
    pltpu.async_copy(act_st, act_out.at[flat_t], sem).wait()
    pltpu.async_copy(rew_in.at[flat_t], f32_st.at[pl.ds(0, _ROWS * T)], sem).wait()
    pltpu.async_copy(f32_st.at[pl.ds(0, _ROWS * T)], rew_out.at[flat_t], sem).wait()
    pltpu.async_copy(lp_in.at[flat_t], f32_st.at[pl.ds(0, _ROWS * T)], sem).wait()
    pltpu.async_copy(f32_st.at[pl.ds(0, _ROWS * T)], lp_out.at[flat_t], sem).wait()
    pltpu.async_copy(val_in.at[flat_v], f32_st, sem).wait()
    pltpu.async_copy(f32_st, val_out.at[flat_v], sem).wait()

    _fill_idx(idx_t, base, T, s_vec)
    _fill_idx(idx_v, base, T + 1, s_vec)

    pltpu.async_copy(a_new.at[rows], upd_i, sem).wait()
    pltpu.async_copy(upd_i, act_out.at[idx_t], sem).wait()
    pltpu.async_copy(r_new.at[rows], upd_v, sem).wait()
    pltpu.async_copy(upd_v, rew_out.at[idx_t], sem).wait()
    pltpu.async_copy(l_new.at[rows], upd_v, sem).wait()
    pltpu.async_copy(upd_v, lp_out.at[idx_t], sem).wait()
    pltpu.async_copy(v_new.at[rows], upd_v, sem).wait()
    pltpu.async_copy(upd_v, val_out.at[idx_v], sem).wait()


_sc_store = functools.partial(
    pl.kernel,
    out_type=(
        jax.ShapeDtypeStruct((B * T,), jnp.int32),
        jax.ShapeDtypeStruct((B * T,), jnp.float32),
        jax.ShapeDtypeStruct((B * T,), jnp.float32),
        jax.ShapeDtypeStruct((B * (T + 1),), jnp.float32),
    ),
    mesh=_vector_mesh,
    scratch_types=[
        pltpu.VMEM((16,), jnp.int32),
        pltpu.VMEM((_ROWS,), jnp.int32),
        pltpu.VMEM((_ROWS,), jnp.int32),
        pltpu.VMEM((_ROWS,), jnp.float32),
        pltpu.VMEM((_ROWS,), jnp.int32),
        pltpu.VMEM((_ROWS * T,), jnp.int32),
        pltpu.VMEM((_ROWS * (T + 1),), jnp.float32),
        pltpu.SemaphoreType.DMA,
    ],
)(_sc_store_body)


def kernel(state_obs, state_action_mask, state_actions, state_rewards,
           state_log_prob, state_values, state_advantages, state_targets,
           step, obs, action_mask, action, reward, log_prob, value):
    step_arr = jnp.asarray(step, jnp.int32).reshape((1,))

    new_act, new_rew, new_lp, new_val = _sc_store(
        state_actions.reshape(B * T),
        state_rewards.reshape(B * T),
        state_log_prob.reshape(B * T),
        state_values.reshape(B * (T + 1)),
        action, reward, log_prob, value,
        jnp.full((16,), jnp.asarray(step, jnp.int32), jnp.int32))

    new_obs, new_mask = pl.pallas_call(
        _tc_kernel,
        grid=(B // _BB,),
        in_specs=[
            pl.BlockSpec(memory_space=pltpu.SMEM),
            pl.BlockSpec((_BB, T, OBS), lambda i: (i, 0, 0)),
            pl.BlockSpec((_BB, T, A), lambda i: (i, 0, 0)),
            pl.BlockSpec((_BB, OBS), lambda i: (i, 0)),
            pl.BlockSpec((_BB, A), lambda i: (i, 0)),
        ],
        out_specs=[
            pl.BlockSpec((_BB, T, OBS), lambda i: (i, 0, 0)),
            pl.BlockSpec((_BB, T, A), lambda i: (i, 0, 0)),
        ],
        out_shape=(
            jax.ShapeDtypeStruct((B, T, OBS), jnp.float32),
            jax.ShapeDtypeStruct((B, T, A), jnp.bool_),
        ),
    )(step_arr, state_obs, state_action_mask, obs, action_mask)

    return (new_obs, new_mask,
            new_act.reshape(B, T),
            new_rew.reshape(B, T),
            new_lp.reshape(B, T),
            new_val.reshape(B, T + 1),
            state_advantages, state_targets)
